# Initial kernel scaffold; baseline (speedup 1.0000x reference)
import jax
import jax.numpy as jnp
from jax.experimental import pallas as pl


def kernel(x):
    raise NotImplementedError("write your pallas kernel here")


# speedup vs baseline: 1.3124x; 1.3124x over previous
import functools

import jax
import jax.numpy as jnp
from jax import lax
from jax.experimental import pallas as pl
from jax.experimental.pallas import tpu as pltpu
from jax.experimental.pallas import tpu_sc as plsc

ROWS = 16384
COLS = 26
CARD = 100
OUT_W = COLS * CARD

NW = 32
ROWS_PER_W = ROWS // NW
G = 32
NGROUPS = ROWS_PER_W // G
XW = ROWS_PER_W * COLS
BUF_W = G * OUT_W


def _onehot_body(x_hbm, out_hbm, xv, buf):
    wid = lax.axis_index("s") * 2 + lax.axis_index("c")
    iota = lax.iota(jnp.int32, 16)
    ones = jnp.full((16,), 1, jnp.int32)
    zeros = jnp.zeros((16,), jnp.int32)

    pltpu.sync_copy(x_hbm.at[pl.ds(wid * XW, XW)], xv)

    def zbody(i, carry):
        for j in range(8):
            buf[pl.ds(i * 128 + j * 16, 16)] = zeros
        return carry
    lax.fori_loop(0, BUF_W // 128, zbody, 0)

    out_base = wid * (ROWS_PER_W * OUT_W)

    def gbody(g, carry):
        x_base = g * (G * COLS)
        pos_list = []
        for sub in range(G // 16):
            row16 = (iota + sub * 16) * OUT_W
            for c in range(COLS):
                gidx = iota * COLS + (x_base + sub * 16 * COLS + c)
                vals = plsc.load_gather(xv, [gidx])
                pos = row16 + (c * CARD) + vals
                plsc.store_scatter(buf, [pos], ones)
                pos_list.append(pos)
        pltpu.sync_copy(buf, out_hbm.at[pl.ds(out_base + g * BUF_W, BUF_W)])
        for pos in pos_list:
            plsc.store_scatter(buf, [pos], zeros)
        return carry

    lax.fori_loop(0, NGROUPS, gbody, 0)


@jax.jit
def kernel(x):
    mesh = plsc.VectorSubcoreMesh(core_axis_name="c", subcore_axis_name="s")
    run = functools.partial(
        pl.kernel,
        mesh=mesh,
        out_type=jax.ShapeDtypeStruct((ROWS * OUT_W,), jnp.int32),
        scratch_types=[
            pltpu.VMEM((XW,), jnp.int32),
            pltpu.VMEM((BUF_W,), jnp.int32),
        ],
        compiler_params=pltpu.CompilerParams(needs_layout_passes=False),
    )(_onehot_body)
    out_flat = run(x.reshape(-1))
    return out_flat.reshape(ROWS, OUT_W)


# speedup vs baseline: 1.3185x; 1.0047x over previous
import functools

import jax
import jax.numpy as jnp
from jax import lax
from jax.experimental import pallas as pl
from jax.experimental.pallas import tpu as pltpu
from jax.experimental.pallas import tpu_sc as plsc

ROWS = 16384
COLS = 26
CARD = 100
OUT_W = COLS * CARD

NW = 32
ROWS_PER_W = ROWS // NW
G = 16
NGROUPS = ROWS_PER_W // G
XW = ROWS_PER_W * COLS
BUF_W = G * OUT_W


def _onehot_body(x_hbm, out_hbm, xv, buf0, buf1, sem0, sem1):
    wid = lax.axis_index("s") * 2 + lax.axis_index("c")
    iota = lax.iota(jnp.int32, 16)
    ones = jnp.full((16,), 1, jnp.int32)
    zeros = jnp.zeros((16,), jnp.int32)

    pltpu.sync_copy(x_hbm.at[pl.ds(wid * XW, XW)], xv)

    def zbody(i, carry):
        for j in range(8):
            buf0[pl.ds(i * 128 + j * 16, 16)] = zeros
            buf1[pl.ds(i * 128 + j * 16, 16)] = zeros
        return carry
    lax.fori_loop(0, BUF_W // 128, zbody, 0)

    out_base = wid * (ROWS_PER_W * OUT_W)

    def scatter(buf, g, value_vec):
        for c in range(COLS):
            gidx = iota * COLS + (g * (G * COLS) + c)
            vals = plsc.load_gather(xv, [gidx])
            pos = iota * OUT_W + (c * CARD) + vals
            plsc.store_scatter(buf, [pos], value_vec)

    def out_slice(g):
        return out_hbm.at[pl.ds(out_base + g * BUF_W, BUF_W)]

    scatter(buf0, 0, ones)
    pltpu.async_copy(buf0, out_slice(0), sem0)
    scatter(buf1, 1, ones)
    pltpu.async_copy(buf1, out_slice(1), sem1)

    def tbody(t, carry):
        for b, (buf, sem) in enumerate(((buf0, sem0), (buf1, sem1))):
            g = 2 * t + b
            pltpu.make_async_copy(buf, out_slice(g - 2), sem).wait()
            scatter(buf, g - 2, zeros)
            scatter(buf, g, ones)
            pltpu.async_copy(buf, out_slice(g), sem)
        return carry
    lax.fori_loop(1, NGROUPS // 2, tbody, 0)

    pltpu.make_async_copy(buf0, out_slice(NGROUPS - 2), sem0).wait()
    pltpu.make_async_copy(buf1, out_slice(NGROUPS - 1), sem1).wait()


@jax.jit
def kernel(x):
    mesh = plsc.VectorSubcoreMesh(core_axis_name="c", subcore_axis_name="s")
    run = functools.partial(
        pl.kernel,
        mesh=mesh,
        out_type=jax.ShapeDtypeStruct((ROWS * OUT_W,), jnp.int32),
        scratch_types=[
            pltpu.VMEM((XW,), jnp.int32),
            pltpu.VMEM((BUF_W,), jnp.int32),
            pltpu.VMEM((BUF_W,), jnp.int32),
            pltpu.SemaphoreType.DMA,
            pltpu.SemaphoreType.DMA,
        ],
        compiler_params=pltpu.CompilerParams(needs_layout_passes=False),
    )(_onehot_body)
    out_flat = run(x.reshape(-1))
    return out_flat.reshape(ROWS, OUT_W)


# speedup vs baseline: 2.0585x; 1.5612x over previous
import functools

import jax
import jax.numpy as jnp
from jax import lax
from jax.experimental import pallas as pl
from jax.experimental.pallas import tpu as pltpu
from jax.experimental.pallas import tpu_sc as plsc

ROWS = 16384
COLS = 26
CARD = 100
OUT_W = COLS * CARD

NW = 32
ROWS_PER_W = ROWS // NW
G = 16
NGROUPS = ROWS_PER_W // G
XW = ROWS_PER_W * COLS


def _onehot_body(x_hbm, z_hbm, out_hbm, xv, buf0, buf1, sem0, sem1):
    wid = lax.axis_index("s") * 2 + lax.axis_index("c")
    iota = lax.iota(jnp.int32, 16)
    ones = jnp.full((16,), 1, jnp.int32)
    zeros = jnp.zeros((16,), jnp.int32)
    row0 = wid * ROWS_PER_W

    pltpu.sync_copy(x_hbm.at[pl.ds(wid * XW, XW)], xv)
    pltpu.sync_copy(z_hbm, buf0)
    pltpu.sync_copy(z_hbm, buf1)

    def scatter(buf, g, value_vec):
        for c in range(COLS):
            gidx = iota * COLS + (g * (G * COLS) + c)
            vals = plsc.load_gather(xv, [gidx])
            col = (c * CARD) + vals
            plsc.store_scatter(buf, [iota, col], value_vec)

    def out_slice(g):
        return out_hbm.at[pl.ds(row0 + g * G, G)]

    scatter(buf0, 0, ones)
    pltpu.async_copy(buf0, out_slice(0), sem0)
    scatter(buf1, 1, ones)
    pltpu.async_copy(buf1, out_slice(1), sem1)

    def tbody(t, carry):
        for b, (buf, sem) in enumerate(((buf0, sem0), (buf1, sem1))):
            g = 2 * t + b
            pltpu.make_async_copy(buf, out_slice(g - 2), sem).wait()
            scatter(buf, g - 2, zeros)
            scatter(buf, g, ones)
            pltpu.async_copy(buf, out_slice(g), sem)
        return carry
    lax.fori_loop(1, NGROUPS // 2, tbody, 0)

    pltpu.make_async_copy(buf0, out_slice(NGROUPS - 2), sem0).wait()
    pltpu.make_async_copy(buf1, out_slice(NGROUPS - 1), sem1).wait()


@jax.jit
def kernel(x):
    mesh = plsc.VectorSubcoreMesh(core_axis_name="c", subcore_axis_name="s")
    run = functools.partial(
        pl.kernel,
        mesh=mesh,
        out_type=jax.ShapeDtypeStruct((ROWS, OUT_W), jnp.int32),
        scratch_types=[
            pltpu.VMEM((XW,), jnp.int32),
            pltpu.VMEM((G, OUT_W), jnp.int32),
            pltpu.VMEM((G, OUT_W), jnp.int32),
            pltpu.SemaphoreType.DMA,
            pltpu.SemaphoreType.DMA,
        ],
        compiler_params=pltpu.CompilerParams(needs_layout_passes=False),
    )(_onehot_body)
    return run(x.reshape(-1), jnp.zeros((G, OUT_W), jnp.int32))


# speedup vs baseline: 5.6000x; 2.7204x over previous
import functools

import jax
import jax.numpy as jnp
from jax import lax
from jax.experimental import pallas as pl
from jax.experimental.pallas import tpu as pltpu
from jax.experimental.pallas import tpu_sc as plsc

ROWS = 16384
COLS = 26
CARD = 100
OUT_W = COLS * CARD

NW = 32
ROWS_PER_W = ROWS // NW
CB = 200
NCB = OUT_W // CB
BR = 256
NB = NCB * 2
XW = ROWS_PER_W * COLS


def _onehot_body(xt_hbm, out_hbm, xv, buf0, buf1, sem0, sem1):
    wid = lax.axis_index("s") * 2 + lax.axis_index("c")
    iota = lax.iota(jnp.int32, 16)
    ones = jnp.full((16,), 1, jnp.int32)
    zeros = jnp.zeros((16,), jnp.int32)
    row0 = wid * ROWS_PER_W

    for c in range(COLS):
        pltpu.sync_copy(
            xt_hbm.at[pl.ds(c * ROWS + row0, ROWS_PER_W)],
            xv.at[pl.ds(c * ROWS_PER_W, ROWS_PER_W)],
        )

    def zbody(k, carry):
        rvec = k * 16 + iota
        for c in range(0, CB, 2):
            for buf in (buf0, buf1):
                plsc.store_scatter(buf, [jnp.full((16,), c, jnp.int32), rvec], zeros)
                plsc.store_scatter(buf, [jnp.full((16,), c + 1, jnp.int32), rvec], zeros)
        return carry
    lax.fori_loop(0, BR // 16, zbody, 0)

    def scatter(buf, q, value_vec):
        cb = q // 2
        hh = q % 2
        for half in range(2):
            xbase = (2 * cb + half) * ROWS_PER_W + hh * BR
            for k in range(BR // 16):
                vals = xv[pl.ds(xbase + k * 16, 16)]
                col = vals + half * CARD
                plsc.store_scatter(buf, [col, k * 16 + iota], value_vec)

    def out_slice(q):
        cb = q // 2
        hh = q % 2
        return out_hbm.at[pl.ds(cb * CB, CB), pl.ds(row0 + hh * BR, BR)]

    scatter(buf0, 0, ones)
    pltpu.async_copy(buf0, out_slice(0), sem0)
    scatter(buf1, 1, ones)
    pltpu.async_copy(buf1, out_slice(1), sem1)

    def tbody(t, carry):
        for b, (buf, sem) in enumerate(((buf0, sem0), (buf1, sem1))):
            q = 2 * t + b
            pltpu.make_async_copy(buf, out_slice(q - 2), sem).wait()
            scatter(buf, q - 2, zeros)
            scatter(buf, q, ones)
            pltpu.async_copy(buf, out_slice(q), sem)
        return carry
    lax.fori_loop(1, NB // 2, tbody, 0)

    pltpu.make_async_copy(buf0, out_slice(NB - 2), sem0).wait()
    pltpu.make_async_copy(buf1, out_slice(NB - 1), sem1).wait()


@jax.jit
def kernel(x):
    mesh = plsc.VectorSubcoreMesh(core_axis_name="c", subcore_axis_name="s")
    run = functools.partial(
        pl.kernel,
        mesh=mesh,
        out_type=jax.ShapeDtypeStruct((OUT_W, ROWS), jnp.int32),
        scratch_types=[
            pltpu.VMEM((XW,), jnp.int32),
            pltpu.VMEM((CB, BR), jnp.int32),
            pltpu.VMEM((CB, BR), jnp.int32),
            pltpu.SemaphoreType.DMA,
            pltpu.SemaphoreType.DMA,
        ],
        compiler_params=pltpu.CompilerParams(needs_layout_passes=False),
    )(_onehot_body)
    out_t = run(x.T.reshape(-1))
    return out_t.T


# speedup vs baseline: 6.5800x; 1.1750x over previous
import functools

import jax
import jax.numpy as jnp
from jax import lax
from jax.experimental import pallas as pl
from jax.experimental.pallas import tpu as pltpu
from jax.experimental.pallas import tpu_sc as plsc

ROWS = 16384
COLS = 26
CARD = 100
OUT_W = COLS * CARD

NW = 32
ROWS_PER_W = ROWS // NW
CB = 200
NCB = OUT_W // CB
BR = 256
NB = NCB * 2
XW = ROWS_PER_W * COLS


def _onehot_body(xt_hbm, out_hbm, xv, buf0, buf1, sem0, sem1, semx):
    wid = lax.axis_index("s") * 2 + lax.axis_index("c")
    iota = lax.iota(jnp.int32, 16)
    ones = jnp.full((16,), 1, jnp.int32)
    zeros = jnp.zeros((16,), jnp.int32)
    row0 = wid * ROWS_PER_W

    def xsrc(c):
        return xt_hbm.at[pl.ds(c * ROWS + row0, ROWS_PER_W)]

    def xdst(c):
        return xv.at[pl.ds(c * ROWS_PER_W, ROWS_PER_W)]

    for c in range(COLS):
        pltpu.async_copy(xsrc(c), xdst(c), semx)

    def make_zbody(buf):
        def zbody(k, carry):
            rvec = k * 16 + iota
            for c in range(0, CB, 2):
                plsc.store_scatter(buf, [jnp.full((16,), c, jnp.int32), rvec], zeros)
                plsc.store_scatter(buf, [jnp.full((16,), c + 1, jnp.int32), rvec], zeros)
            return carry
        return zbody

    lax.fori_loop(0, BR // 16, make_zbody(buf0), 0)
    for c in range(COLS):
        pltpu.make_async_copy(xsrc(c), xdst(c), semx).wait()

    def scatter(buf, q, value_vec):
        cb = q // 2
        hh = q % 2
        for half in range(2):
            xbase = (2 * cb + half) * ROWS_PER_W + hh * BR
            for k in range(BR // 16):
                vals = xv[pl.ds(xbase + k * 16, 16)]
                col = vals + half * CARD
                plsc.store_scatter(buf, [col, k * 16 + iota], value_vec)

    def out_slice(q):
        cb = q // 2
        hh = q % 2
        return out_hbm.at[pl.ds(cb * CB, CB), pl.ds(row0 + hh * BR, BR)]

    scatter(buf0, 0, ones)
    pltpu.async_copy(buf0, out_slice(0), sem0)
    lax.fori_loop(0, BR // 16, make_zbody(buf1), 0)
    scatter(buf1, 1, ones)
    pltpu.async_copy(buf1, out_slice(1), sem1)

    def tbody(t, carry):
        for b, (buf, sem) in enumerate(((buf0, sem0), (buf1, sem1))):
            q = 2 * t + b
            pltpu.make_async_copy(buf, out_slice(q - 2), sem).wait()
            scatter(buf, q - 2, zeros)
            scatter(buf, q, ones)
            pltpu.async_copy(buf, out_slice(q), sem)
        return carry
    lax.fori_loop(1, NB // 2, tbody, 0)

    pltpu.make_async_copy(buf0, out_slice(NB - 2), sem0).wait()
    pltpu.make_async_copy(buf1, out_slice(NB - 1), sem1).wait()


@jax.jit
def kernel(x):
    mesh = plsc.VectorSubcoreMesh(core_axis_name="c", subcore_axis_name="s")
    run = functools.partial(
        pl.kernel,
        mesh=mesh,
        out_type=jax.ShapeDtypeStruct((OUT_W, ROWS), jnp.int32),
        scratch_types=[
            pltpu.VMEM((XW,), jnp.int32),
            pltpu.VMEM((CB, BR), jnp.int32),
            pltpu.VMEM((CB, BR), jnp.int32),
            pltpu.SemaphoreType.DMA,
            pltpu.SemaphoreType.DMA,
            pltpu.SemaphoreType.DMA,
        ],
        compiler_params=pltpu.CompilerParams(needs_layout_passes=False),
    )(_onehot_body)
    out_t = run(x.T.reshape(-1))
    return out_t.T
